# Initial kernel scaffold; baseline (speedup 1.0000x reference)
import jax
import jax.numpy as jnp
from jax.experimental import pallas as pl


def kernel(phenotypes, prototypes, edge_index, clinical_features, W1l, W1r, b1, W2l, W2r, b2, Wx, bx, Wc, bc, Ws, bs, Wfc, bfc):
    raise NotImplementedError("write your pallas kernel here")


# speedup vs baseline: 11.3477x; 11.3477x over previous
import functools

import jax
import jax.numpy as jnp
from jax import lax
from jax.experimental import pallas as pl
from jax.experimental.pallas import tpu as pltpu, tpu_sc as plsc

N = 10000
E = 160000
D = 256
H = 256
A = 64
C = 16
NC = 2

NP = 10240
NSUB = 16
NCORE = 2
RPT = NP // NSUB
CH = 80
EPT = E // NSUB
TCHUNKS = EPT // CH

f32 = jnp.float32

_sc_mesh = plsc.VectorSubcoreMesh(core_axis_name="c", subcore_axis_name="s",
                                  num_cores=NCORE, num_subcores=NSUB)
_no_layout = pltpu.CompilerParams(needs_layout_passes=False)


def _zero_1d(ref, n):
    def z(i, _):
        ref[pl.ds(i * 16, 16)] = jnp.zeros((16,), f32)
        return 0
    lax.fori_loop(0, n // 16, z, 0)


def _pipe_loop(tbl, src1d, dst1d, agg_sh, tb, start, count,
               SRC, DST, ROWS, SS, DS, GS, on_chunk=None):
    last = start + count - 1

    def stage(b, c):
        off = tb + jnp.minimum(c, last) * CH
        pltpu.async_copy(src1d.at[pl.ds(off, CH)], SRC[b], SS[b])
        pltpu.async_copy(dst1d.at[pl.ds(off, CH)], DST[b], DS[b])

    def wait_stage(b):
        pltpu.make_async_copy(src1d.at[pl.ds(0, CH)], SRC[b], SS[b]).wait()
        pltpu.make_async_copy(dst1d.at[pl.ds(0, CH)], DST[b], DS[b]).wait()

    def gather(b):
        pltpu.async_copy(tbl.at[SRC[b]], ROWS[b], GS[b])

    def wait_gather(b):
        pltpu.make_async_copy(tbl.at[pl.ds(0, CH)], ROWS[b], GS[b]).wait()

    def scatter(b):
        pltpu.sync_copy(ROWS[b], agg_sh.at[DST[b]], add=True)
        if on_chunk is not None:
            on_chunk(b)

    stage(0, start)
    wait_stage(0)
    gather(0)
    stage(1, start + 1)

    def body(p, _):
        c0 = start + 2 * p
        wait_gather(0)
        wait_stage(1)
        gather(1)
        scatter(0)
        stage(0, c0 + 2)
        wait_gather(1)
        wait_stage(0)
        gather(0)
        scatter(1)
        stage(1, c0 + 3)
        return 0
    lax.fori_loop(0, count // 2, body, 0)

    wait_gather(0)
    if count % 2:
        scatter(0)
    wait_stage(1)


@functools.partial(
    pl.kernel,
    out_type=[
        jax.ShapeDtypeStruct((NP, 128), f32),
        jax.ShapeDtypeStruct((NP, 128), f32),
        jax.ShapeDtypeStruct((NSUB * NP,), f32),
    ],
    mesh=_sc_mesh,
    scratch_types=[
        pltpu.VMEM((CH,), jnp.int32),
        pltpu.VMEM((CH,), jnp.int32),
        pltpu.VMEM((CH,), jnp.int32),
        pltpu.VMEM((CH,), jnp.int32),
        pltpu.VMEM((CH, 128), f32),
        pltpu.VMEM((CH, 128), f32),
        pltpu.VMEM((NP,), f32),
        pltpu.VMEM_SHARED((NP, 128), f32),
        pltpu.SemaphoreType.DMA,
        pltpu.SemaphoreType.DMA,
        pltpu.SemaphoreType.DMA,
        pltpu.SemaphoreType.DMA,
        pltpu.SemaphoreType.DMA,
        pltpu.SemaphoreType.DMA,
    ],
    compiler_params=_no_layout,
)
def _sc_edge_agg(y_lo, y_hi, src1d, dst1d, zeros128, agg_lo_out, agg_hi_out,
                 deg_out, src_v0, src_v1, dst_v0, dst_v1, rows_v0, rows_v1,
                 deg_v, agg_sh, ssem0, ssem1, dsem0, dsem1, gsem0, gsem1):
    cid = lax.axis_index("c")
    sid = lax.axis_index("s")
    r0 = sid * RPT
    tb = sid * EPT

    pltpu.sync_copy(zeros128, rows_v0)

    def z(k, _):
        pltpu.sync_copy(rows_v0, agg_sh.at[pl.ds(r0 + k * CH, CH)])
        return 0
    lax.fori_loop(0, RPT // CH, z, 0)
    _zero_1d(deg_v, NP)
    plsc.subcore_barrier()

    DST = [dst_v0, dst_v1]
    kw = dict(SRC=[src_v0, src_v1], DST=DST, ROWS=[rows_v0, rows_v1],
              SS=[ssem0, ssem1], DS=[dsem0, dsem1], GS=[gsem0, gsem1])
    ones16 = jnp.full((16,), 1.0, f32)

    def deg_add(b):
        for k in range(CH // 16):
            di = DST[b][pl.ds(k * 16, 16)]
            plsc.addupdate_scatter(deg_v, [di], ones16)

    @pl.when(cid == 0)
    def _():
        _pipe_loop(y_lo, src1d, dst1d, agg_sh, tb, 0, TCHUNKS,
                   on_chunk=deg_add, **kw)

    @pl.when(cid == 1)
    def _():
        _pipe_loop(y_hi, src1d, dst1d, agg_sh, tb, 0, TCHUNKS, **kw)

    plsc.subcore_barrier()

    @pl.when(cid == 0)
    def _():
        pltpu.sync_copy(agg_sh.at[pl.ds(r0, RPT)], agg_lo_out.at[pl.ds(r0, RPT)])
        pltpu.sync_copy(deg_v, deg_out.at[pl.ds(sid * NP, NP)])

    @pl.when(cid == 1)
    def _():
        pltpu.sync_copy(agg_sh.at[pl.ds(r0, RPT)], agg_hi_out.at[pl.ds(r0, RPT)])


@functools.partial(
    pl.kernel,
    out_type=jax.ShapeDtypeStruct((NSUB * NP,), f32),
    mesh=_sc_mesh,
    scratch_types=[
        pltpu.VMEM((NP,), f32),
        pltpu.VMEM((NP,), f32),
        pltpu.VMEM((EPT,), jnp.int32),
        pltpu.VMEM((EPT,), jnp.int32),
    ],
    compiler_params=_no_layout,
)
def _sc_scalar_agg(s_tbl, src1d, dst1d, aggs_out, tbl_v, acc_v, src_v, dst_v):
    cid = lax.axis_index("c")
    sid = lax.axis_index("s")

    @pl.when(cid == 0)
    def _():
        tb = sid * EPT
        _zero_1d(acc_v, NP)
        pltpu.sync_copy(s_tbl, tbl_v)
        pltpu.sync_copy(src1d.at[pl.ds(tb, EPT)], src_v)
        pltpu.sync_copy(dst1d.at[pl.ds(tb, EPT)], dst_v)

        def step(k, _):
            si = src_v[pl.ds(k * 16, 16)]
            di = dst_v[pl.ds(k * 16, 16)]
            vals = plsc.load_gather(tbl_v, [si])
            plsc.addupdate_scatter(acc_v, [di], vals)
            return 0
        lax.fori_loop(0, EPT // 16, step, 0)
        pltpu.sync_copy(acc_v, aggs_out.at[pl.ds(sid * NP, NP)])


_R = 1280
_G = NP // _R


def _tc1_body(ph, pr, wl, wr, ylo, yhi, z):
    a = ph[...]
    b = jnp.abs(a - pr[...])
    y = (jnp.dot(a, wl[:D, :], preferred_element_type=f32)
         + jnp.dot(b, wl[D:, :], preferred_element_type=f32))
    ylo[...] = y[:, :128]
    yhi[...] = y[:, 128:]
    z[...] = (jnp.dot(a, wr[:D, :], preferred_element_type=f32)
              + jnp.dot(b, wr[D:, :], preferred_element_type=f32))


def _tc2_body(agg_lo, agg_hi, degp, z, b1r, w2, x1o, so):
    dg = jnp.sum(degp[...], axis=0)[:, None]
    inv = 1.0 / jnp.maximum(dg, 1.0)
    mean = jnp.concatenate([agg_lo[...], agg_hi[...]], axis=1) * inv
    x1 = jnp.maximum(mean + z[...] + b1r[...], 0.0)
    x1o[...] = x1
    so[...] = jnp.dot(x1, w2[...], preferred_element_type=f32)


def _tc3_body(x1, s, aggsp, degp, clin, wx, bx, wc2, bc,
              ws2, bs, wfc, bfc, b2, out_o, aw_o):
    ws2a = ws2[...]
    alpha = jnp.sum(ws2a[:, :A] * wx[...], axis=1)
    cterm = jnp.reshape(clin[...], (C, 1)) * wc2[...] + bc[...]
    beta = (jnp.sum(bx[...] * ws2a[:, :A], axis=1)
            + jnp.sum(cterm * ws2a[:, A:], axis=1)
            + bs[...][:, 0])

    dgc = jnp.maximum(jnp.sum(degp[...], axis=0)[:, None], 1.0)
    agg = jnp.sum(aggsp[...], axis=0)[:, None]
    t = agg / dgc + s[...][:, 1:2] + b2[...]
    aw = jax.nn.sigmoid(t * alpha[None, :] + beta[None, :])
    aw_o[...] = aw

    rows = lax.broadcasted_iota(jnp.int32, (NP, 1), 0)
    valid = rows < N
    r = jnp.sum(aw, axis=1, keepdims=True)
    rm = jnp.where(valid, r, -jnp.inf)
    e = jnp.where(valid, jnp.exp(rm - jnp.max(rm)), 0.0)
    w = e / jnp.sum(e)
    xp = lax.dot_general(w, x1[...], (((0,), (0,)), ((), ())),
                         preferred_element_type=f32)
    wfca = wfc[...]
    logits = (jnp.dot(xp, wfca[:H, :], preferred_element_type=f32)
              + jnp.dot(clin[...], wfca[H:, :], preferred_element_type=f32)
              + bfc[...])
    out_o[...] = jax.nn.sigmoid(logits)


def kernel(phenotypes, prototypes, edge_index, clinical_features,
           W1l, W1r, b1, W2l, W2r, b2, Wx, bx, Wc, bc, Ws, bs, Wfc, bfc):
    pad = ((0, NP - N), (0, 0))
    phen_p = jnp.pad(phenotypes, pad)
    prot_p = jnp.pad(prototypes, pad)
    src1d = edge_index[0]
    dst1d = edge_index[1]
    zeros128 = jnp.zeros((CH, 128), f32)
    w2cat = jnp.pad(jnp.concatenate([W2l, W2r], axis=1), ((0, 0), (0, 14)))
    ws2 = Ws.reshape(C, 2 * A)
    wc2 = Wc.reshape(C, A)
    b1r = b1.reshape(1, H)
    b2r = b2.reshape(1, 1)
    bxr = bx.reshape(1, A)
    bfcr = bfc.reshape(1, NC)

    full = lambda shp: pl.BlockSpec(shp, lambda i: (0,) * len(shp))
    rowblk = lambda cols: pl.BlockSpec((_R, cols), lambda i: (i, 0))

    y_lo, y_hi, z = pl.pallas_call(
        _tc1_body,
        grid=(_G,),
        in_specs=[rowblk(D), rowblk(D), full((2 * D, H)), full((2 * D, H))],
        out_specs=[rowblk(128), rowblk(128), rowblk(H)],
        out_shape=[
            jax.ShapeDtypeStruct((NP, 128), f32),
            jax.ShapeDtypeStruct((NP, 128), f32),
            jax.ShapeDtypeStruct((NP, H), f32),
        ],
    )(phen_p, prot_p, W1l, W1r)

    agg_lo, agg_hi, deg_flat = _sc_edge_agg(y_lo, y_hi, src1d, dst1d,
                                            zeros128)
    degp = deg_flat.reshape(NSUB, NP)

    x1, s = pl.pallas_call(
        _tc2_body,
        grid=(_G,),
        in_specs=[rowblk(128), rowblk(128),
                  pl.BlockSpec((NSUB, _R), lambda i: (0, i)),
                  rowblk(H), full((1, H)), full((H, 16))],
        out_specs=[rowblk(H), rowblk(16)],
        out_shape=[
            jax.ShapeDtypeStruct((NP, H), f32),
            jax.ShapeDtypeStruct((NP, 16), f32),
        ],
    )(agg_lo, agg_hi, degp, z, b1r, w2cat)

    s_l = s[:, 0]
    aggs_flat = _sc_scalar_agg(s_l, src1d, dst1d)
    aggsp = aggs_flat.reshape(NSUB, NP)

    out, aw = pl.pallas_call(
        _tc3_body,
        out_shape=[
            jax.ShapeDtypeStruct((1, NC), f32),
            jax.ShapeDtypeStruct((NP, C), f32),
        ],
    )(x1, s, aggsp, degp, clinical_features, Wx, bxr, wc2,
      bc, ws2, bs, Wfc, bfcr, b2r)

    return (out, aw[:N].reshape(N, C, 1))
